# baseline (speedup 1.0000x reference)
import jax
import jax.numpy as jnp
from jax import lax
from jax.experimental import pallas as pl
from jax.experimental.pallas import tpu as pltpu
from jax.experimental.pallas import tpu_sc as plsc

B, N, P, D = 64, 16384, 64, 34
L = 16
NC, NS = 2, 16
NW = NC * NS
IPW = B // NW
PG = P // L


def _image_loss(gts_v, preds_v):
    zero = jnp.zeros((L,), jnp.float32)
    init = (tuple(zero for _ in range(PG)), tuple(zero for _ in range(PG)))

    @plsc.parallel_loop(0, D, carry=init, unroll=4)
    def final(d, carry):
        errs, vaccs = carry
        r = d * 3
        ne, nv = [], []
        for g in range(PG):
            s = pl.ds(g * L, L)
            val = gts_v[r, s]
            pif = gts_v[r + 1, s]
            flg = gts_v[r + 2, s]
            prd = plsc.load_gather(preds_v, [pif.astype(jnp.int32)])
            m = flg > 0.0
            ne.append(errs[g] + jnp.where(m, jnp.abs(prd - val), 0.0))
            nv.append(jnp.maximum(vaccs[g], flg))
        return tuple(ne), tuple(nv)

    errs, vaccs = final
    npeople = jnp.zeros((L,), jnp.int32)
    for g in range(PG):
        npeople = npeople + plsc.all_reduce_population_count(vaccs[g] > 0.0)
    total = jnp.sum(errs[0] + errs[1] + errs[2] + errs[3])
    return total / jnp.maximum(npeople.astype(jnp.float32), 1.0)


def _body(preds_hbm, gts_hbm, out_hbm,
          preds_v0, gts_v0, preds_v1, gts_v1, res_v, sem):
    wid = lax.axis_index("s") * NC + lax.axis_index("c")
    i0 = wid * IPW
    lane = lax.iota(jnp.int32, L)
    cp0 = pltpu.async_copy(preds_hbm.at[i0], preds_v0, sem)
    cg0 = pltpu.async_copy(gts_hbm.at[:, i0], gts_v0, sem)
    cp1 = pltpu.async_copy(preds_hbm.at[i0 + 1], preds_v1, sem)
    cg1 = pltpu.async_copy(gts_hbm.at[:, i0 + 1], gts_v1, sem)
    cp0.wait()
    cg0.wait()
    loss0 = _image_loss(gts_v0, preds_v0)
    cp1.wait()
    cg1.wait()
    loss1 = _image_loss(gts_v1, preds_v1)
    res = jnp.where(lane == 0, loss0, jnp.where(lane == 1, loss1, 0.0))
    res_v[...] = res
    pltpu.sync_copy(res_v, out_hbm.at[wid])


def kernel(preds, gts):
    gts_t = jnp.transpose(gts, (2, 3, 0, 1)).reshape(D * 3, B, P)
    mesh = plsc.VectorSubcoreMesh(core_axis_name="c", subcore_axis_name="s")
    f = pl.kernel(
        _body,
        mesh=mesh,
        out_type=jax.ShapeDtypeStruct((NW, L), jnp.float32),
        scratch_types=[
            pltpu.VMEM((N,), jnp.float32),
            pltpu.VMEM((D * 3, P), jnp.float32),
            pltpu.VMEM((N,), jnp.float32),
            pltpu.VMEM((D * 3, P), jnp.float32),
            pltpu.VMEM((L,), jnp.float32),
            pltpu.SemaphoreType.DMA,
        ],
        compiler_params=pltpu.CompilerParams(
            needs_layout_passes=False,
            skip_device_barrier=True,
        ),
    )
    out2d = f(preds, gts_t)
    return out2d[:, :IPW].reshape(B)


# speedup vs baseline: 1.0234x; 1.0234x over previous
import jax
import jax.numpy as jnp
from jax import lax
from jax.experimental import pallas as pl
from jax.experimental.pallas import tpu as pltpu
from jax.experimental.pallas import tpu_sc as plsc

B, N, P, D = 64, 16384, 64, 34
L = 16
NC, NS = 2, 16
NW = NC * NS
IPW = B // NW
PG = P // L


def _image_loss(gts_v, preds_v):
    zero = jnp.zeros((L,), jnp.float32)
    init = (tuple(zero for _ in range(PG)), tuple(zero for _ in range(PG)))

    @plsc.parallel_loop(0, D, carry=init, unroll=2)
    def final(d, carry):
        errs, vaccs = carry
        r = d * 3
        ne, nv = [], []
        for g in range(PG):
            s = pl.ds(g * L, L)
            val = gts_v[r, s]
            pif = gts_v[r + 1, s]
            flg = gts_v[r + 2, s]
            prd = plsc.load_gather(preds_v, [pif.astype(jnp.int32)])
            m = flg > 0.0
            ne.append(errs[g] + jnp.where(m, jnp.abs(prd - val), 0.0))
            nv.append(jnp.maximum(vaccs[g], flg))
        return tuple(ne), tuple(nv)

    errs, vaccs = final
    npeople = jnp.zeros((L,), jnp.int32)
    for g in range(PG):
        npeople = npeople + plsc.all_reduce_population_count(vaccs[g] > 0.0)
    total = jnp.sum(errs[0] + errs[1] + errs[2] + errs[3])
    return total / jnp.maximum(npeople.astype(jnp.float32), 1.0)


def _body(preds_hbm, gts_hbm, out_hbm,
          preds_v0, gts_v0, preds_v1, gts_v1, res_v, sem):
    wid = lax.axis_index("s") * NC + lax.axis_index("c")
    i0 = wid * IPW
    lane = lax.iota(jnp.int32, L)
    cp0 = pltpu.async_copy(preds_hbm.at[i0], preds_v0, sem)
    cg0 = pltpu.async_copy(gts_hbm.at[:, i0], gts_v0, sem)
    cp1 = pltpu.async_copy(preds_hbm.at[i0 + 1], preds_v1, sem)
    cg1 = pltpu.async_copy(gts_hbm.at[:, i0 + 1], gts_v1, sem)
    cp0.wait()
    cg0.wait()
    loss0 = _image_loss(gts_v0, preds_v0)
    cp1.wait()
    cg1.wait()
    loss1 = _image_loss(gts_v1, preds_v1)
    res = jnp.where(lane == 0, loss0, jnp.where(lane == 1, loss1, 0.0))
    res_v[...] = res
    pltpu.sync_copy(res_v, out_hbm.at[wid])


def kernel(preds, gts):
    gts_t = jnp.transpose(gts, (2, 3, 0, 1)).reshape(D * 3, B, P)
    mesh = plsc.VectorSubcoreMesh(core_axis_name="c", subcore_axis_name="s")
    f = pl.kernel(
        _body,
        mesh=mesh,
        out_type=jax.ShapeDtypeStruct((NW, L), jnp.float32),
        scratch_types=[
            pltpu.VMEM((N,), jnp.float32),
            pltpu.VMEM((D * 3, P), jnp.float32),
            pltpu.VMEM((N,), jnp.float32),
            pltpu.VMEM((D * 3, P), jnp.float32),
            pltpu.VMEM((L,), jnp.float32),
            pltpu.SemaphoreType.DMA,
        ],
        compiler_params=pltpu.CompilerParams(
            needs_layout_passes=False,
            skip_device_barrier=True,
        ),
    )
    out2d = f(preds, gts_t)
    return out2d[:, :IPW].reshape(B)


# speedup vs baseline: 1.0655x; 1.0411x over previous
import jax
import jax.numpy as jnp
from jax import lax
from jax.experimental import pallas as pl
from jax.experimental.pallas import tpu as pltpu
from jax.experimental.pallas import tpu_sc as plsc

B, N, P, D = 64, 16384, 64, 34
L = 16
NC, NS = 2, 16
NW = NC * NS
IPW = B // NW
PG = P // L


def _image_loss(gts_v, preds_v):
    zero = jnp.zeros((L,), jnp.float32)
    init = (tuple(zero for _ in range(PG)), tuple(zero for _ in range(PG)))

    @plsc.parallel_loop(0, D, carry=init, unroll=2)
    def final(d, carry):
        errs, vaccs = carry
        r = d * 3
        ne, nv = [], []
        for g in range(PG):
            s = pl.ds(g * L, L)
            val = gts_v[r, s]
            pif = gts_v[r + 1, s]
            flg = gts_v[r + 2, s]
            prd = plsc.load_gather(preds_v, [pif.astype(jnp.int32)])
            m = flg > 0.0
            ne.append(errs[g] + jnp.where(m, jnp.abs(prd - val), 0.0))
            nv.append(jnp.maximum(vaccs[g], flg))
        return tuple(ne), tuple(nv)

    errs, vaccs = final
    npeople = jnp.zeros((L,), jnp.int32)
    for g in range(PG):
        npeople = npeople + plsc.all_reduce_population_count(vaccs[g] > 0.0)
    total = jnp.sum(errs[0] + errs[1] + errs[2] + errs[3])
    return total / jnp.maximum(npeople.astype(jnp.float32), 1.0)


def _body(preds_hbm, gts_hbm, out_hbm,
          preds_v0, gts_v0, preds_v1, gts_v1,
          res_v, shared_res, pack_v, out_v, sem):
    cid = lax.axis_index("c")
    sid = lax.axis_index("s")
    wid = cid * NS + sid
    i0 = wid * IPW
    lane = lax.iota(jnp.int32, L)
    cp0 = pltpu.async_copy(preds_hbm.at[i0], preds_v0, sem)
    cg0 = pltpu.async_copy(gts_hbm.at[:, i0], gts_v0, sem)
    cp1 = pltpu.async_copy(preds_hbm.at[i0 + 1], preds_v1, sem)
    cg1 = pltpu.async_copy(gts_hbm.at[:, i0 + 1], gts_v1, sem)
    cp0.wait()
    cg0.wait()
    loss0 = _image_loss(gts_v0, preds_v0)
    cp1.wait()
    cg1.wait()
    loss1 = _image_loss(gts_v1, preds_v1)
    res = jnp.where(lane == 0, loss0, jnp.where(lane == 1, loss1, 0.0))
    res_v[...] = res
    pltpu.sync_copy(res_v, shared_res.at[sid])
    plsc.subcore_barrier()

    @pl.when(sid == 0)
    def _():
        pltpu.sync_copy(shared_res, pack_v)
        for k in range(2):
            flat = k * L + lane
            vals = plsc.load_gather(
                pack_v, [flat >> 1, flat & 1])
            out_v[pl.ds(k * L, L)] = vals
        pltpu.sync_copy(out_v, out_hbm.at[pl.ds(cid * (NS * IPW), NS * IPW)])


def kernel(preds, gts):
    gts_t = jnp.transpose(gts, (2, 3, 0, 1)).reshape(D * 3, B, P)
    mesh = plsc.VectorSubcoreMesh(core_axis_name="c", subcore_axis_name="s")
    f = pl.kernel(
        _body,
        mesh=mesh,
        out_type=jax.ShapeDtypeStruct((B,), jnp.float32),
        scratch_types=[
            pltpu.VMEM((N,), jnp.float32),
            pltpu.VMEM((D * 3, P), jnp.float32),
            pltpu.VMEM((N,), jnp.float32),
            pltpu.VMEM((D * 3, P), jnp.float32),
            pltpu.VMEM((L,), jnp.float32),
            pltpu.VMEM_SHARED((NS, L), jnp.float32),
            pltpu.VMEM((NS, L), jnp.float32),
            pltpu.VMEM((NS * IPW,), jnp.float32),
            pltpu.SemaphoreType.DMA,
        ],
        compiler_params=pltpu.CompilerParams(
            needs_layout_passes=False,
            skip_device_barrier=True,
        ),
    )
    return f(preds, gts_t)


# speedup vs baseline: 1.0778x; 1.0116x over previous
import jax
import jax.numpy as jnp
from jax import lax
from jax.experimental import pallas as pl
from jax.experimental.pallas import tpu as pltpu
from jax.experimental.pallas import tpu_sc as plsc

B, N, P, D = 64, 16384, 64, 34
L = 16
NC, NS = 2, 16
NW = NC * NS
IPW = B // NW
PG = P // L


def _image_loss(gts_v, preds_v):
    zero = jnp.zeros((L,), jnp.float32)
    init = (tuple(zero for _ in range(PG)), tuple(zero for _ in range(PG)))

    @plsc.parallel_loop(0, D, carry=init, unroll=2)
    def final(d, carry):
        errs, vaccs = carry
        r = d * 3
        ne, nv = [], []
        for g in range(PG):
            s = pl.ds(g * L, L)
            val = gts_v[r, s]
            pif = gts_v[r + 1, s]
            flg = gts_v[r + 2, s]
            prd = plsc.load_gather(preds_v, [pif.astype(jnp.int32)])
            m = flg > 0.0
            ne.append(errs[g] + jnp.where(m, jnp.abs(prd - val), 0.0))
            nv.append(jnp.maximum(vaccs[g], flg))
        return tuple(ne), tuple(nv)

    errs, vaccs = final
    npeople = jnp.zeros((L,), jnp.int32)
    for g in range(PG):
        npeople = npeople + plsc.all_reduce_population_count(vaccs[g] > 0.0)
    total = jnp.sum(errs[0] + errs[1] + errs[2] + errs[3])
    return total / jnp.maximum(npeople.astype(jnp.float32), 1.0)


def _body(preds_hbm, gts_hbm, out_hbm,
          preds_v0, gts_v0, preds_v1, gts_v1,
          res_v, shared_res, pack_v, out_v, sem):
    cid = lax.axis_index("c")
    sid = lax.axis_index("s")
    wid = cid * NS + sid
    i0 = wid * IPW
    lane = lax.iota(jnp.int32, L)
    cp0 = pltpu.async_copy(preds_hbm.at[i0], preds_v0, sem)
    cg0 = pltpu.async_copy(gts_hbm.at[:, i0], gts_v0, sem)
    cp1 = pltpu.async_copy(preds_hbm.at[i0 + 1], preds_v1, sem)
    cg1 = pltpu.async_copy(gts_hbm.at[:, i0 + 1], gts_v1, sem)
    cp0.wait()
    cg0.wait()
    loss0 = _image_loss(gts_v0, preds_v0)
    cp1.wait()
    cg1.wait()
    loss1 = _image_loss(gts_v1, preds_v1)
    l0 = (2 * sid) % L
    res = jnp.where(lane == l0, loss0, jnp.where(lane == l0 + 1, loss1, 0.0))
    res_v[...] = res
    pltpu.sync_copy(res_v, shared_res.at[pl.ds(sid * L, L)])
    plsc.subcore_barrier()

    @pl.when(sid == 0)
    def _():
        pltpu.sync_copy(shared_res, pack_v)
        for k in range(2):
            acc = jnp.zeros((L,), jnp.float32)
            for s in range(NS // 2):
                acc = acc + pack_v[pl.ds((k * (NS // 2) + s) * L, L)]
            out_v[pl.ds(k * L, L)] = acc
        pltpu.sync_copy(out_v, out_hbm.at[pl.ds(cid * (NS * IPW), NS * IPW)])


def kernel(preds, gts):
    gts_t = jnp.transpose(gts, (2, 3, 0, 1)).reshape(D * 3, B, P)
    mesh = plsc.VectorSubcoreMesh(core_axis_name="c", subcore_axis_name="s")
    f = pl.kernel(
        _body,
        mesh=mesh,
        out_type=jax.ShapeDtypeStruct((B,), jnp.float32),
        scratch_types=[
            pltpu.VMEM((N,), jnp.float32),
            pltpu.VMEM((D * 3, P), jnp.float32),
            pltpu.VMEM((N,), jnp.float32),
            pltpu.VMEM((D * 3, P), jnp.float32),
            pltpu.VMEM((L,), jnp.float32),
            pltpu.VMEM_SHARED((NS * L,), jnp.float32),
            pltpu.VMEM((NS * L,), jnp.float32),
            pltpu.VMEM((NS * IPW,), jnp.float32),
            pltpu.SemaphoreType.DMA,
        ],
        compiler_params=pltpu.CompilerParams(
            needs_layout_passes=False,
            skip_device_barrier=True,
        ),
    )
    return f(preds, gts_t)
